# baseline (speedup 1.0000x reference)
import functools

import jax
import jax.numpy as jnp
from jax import lax
from jax.experimental import pallas as pl
from jax.experimental.pallas import tpu as pltpu
from jax.experimental.pallas import tpu_sc as plsc

NC = 2
NS = 16
NW = NC * NS
CHUNK = 128
DEG_W = 16
PIPE_G = 3


def _sc_mesh():
    return plsc.VectorSubcoreMesh(
        core_axis_name="c", subcore_axis_name="s", num_cores=NC, num_subcores=NS
    )


def _deg_body(dst_hbm, zeros_hbm, ones_hbm, deg_out, idx_v, ones_v, *scratch,
              n, n_pad, chunks):
    sems = scratch[:PIPE_G]
    deg_sh = scratch[PIPE_G]
    c = lax.axis_index("c")
    s = lax.axis_index("s")
    w = s * NC + c
    rpt = n_pad // NS
    pltpu.sync_copy(zeros_hbm, deg_sh.at[pl.ds(s * rpt, rpt)])
    pltpu.sync_copy(dst_hbm.at[w], idx_v)
    pltpu.sync_copy(ones_hbm, ones_v)
    plsc.subcore_barrier()

    ng = chunks // PIPE_G
    rem = chunks - ng * PIPE_G

    def group(base, count):
        descs = []
        for i in range(count):
            descs.append(
                pltpu.async_copy(ones_v, deg_sh.at[idx_v.at[base + i]],
                                 sems[i], add=True)
            )
        for i in range(count):
            descs[i].wait()

    def body(g, carry):
        group(g * PIPE_G, PIPE_G)
        return carry

    lax.fori_loop(0, ng, body, 0)
    if rem:
        group(ng * PIPE_G, rem)
    plsc.subcore_barrier()
    pltpu.sync_copy(deg_sh.at[pl.ds(s * rpt, rpt)],
                    deg_out.at[c, pl.ds(s * rpt, rpt)])


def _edge_body(src_hbm, dst_hbm, y_hbm, zeros_hbm, acc_out,
               src_v, dst_v, *scratch, n, n_pad, chunks, hid):
    rows = scratch[:PIPE_G]
    sems = scratch[PIPE_G:2 * PIPE_G]
    y_sh, acc_sh = scratch[2 * PIPE_G:]
    c = lax.axis_index("c")
    s = lax.axis_index("s")
    w = s * NC + c
    rpt = n_pad // NS
    spt = n // NS
    pltpu.sync_copy(y_hbm.at[pl.ds(s * spt, spt)], y_sh.at[pl.ds(s * spt, spt)])

    @pl.when(c == 0)
    def _():
        pltpu.sync_copy(y_hbm.at[pl.ds(s * spt, spt)],
                        acc_sh.at[pl.ds(s * spt, spt)])

    @pl.when(c != 0)
    def _():
        pltpu.sync_copy(zeros_hbm, acc_sh.at[pl.ds(s * rpt, rpt)])

    pltpu.sync_copy(src_hbm.at[w], src_v)
    pltpu.sync_copy(dst_hbm.at[w], dst_v)
    plsc.subcore_barrier()

    ng = chunks // PIPE_G
    rem = chunks - ng * PIPE_G

    def group(base, count):
        gd = []
        for i in range(count):
            gd.append(
                pltpu.async_copy(y_sh.at[src_v.at[base + i]], rows[i], sems[i])
            )
        for i in range(count):
            gd[i].wait()
            pltpu.sync_copy(rows[i], acc_sh.at[dst_v.at[base + i]], add=True)

    def body(g, carry):
        group(g * PIPE_G, PIPE_G)
        return carry

    lax.fori_loop(0, ng, body, 0)
    if rem:
        group(ng * PIPE_G, rem)
    plsc.subcore_barrier()
    pltpu.sync_copy(acc_sh.at[pl.ds(s * rpt, rpt)],
                    acc_out.at[c, pl.ds(s * rpt, rpt)])


def _tc_a_body(x_ref, degp_ref, wg_ref, wps_ref, bps_ref, y_ref, zsem_ref):
    d = degp_ref[...]
    deg = d[0, :, :1] + d[1, :, :1] + 1.0
    dinv = lax.rsqrt(deg)
    xw = jnp.dot(x_ref[...], wg_ref[...], preferred_element_type=jnp.float32)
    y_ref[...] = xw * dinv
    zsem_ref[...] = (
        jnp.dot(x_ref[...], wps_ref[...], preferred_element_type=jnp.float32)
        + bps_ref[...]
    )


def _tc_e_body(acc_ref, degp_ref, zsem_ref, bg_ref, wpt_ref, bpt_ref,
               wcls_ref, bcls_ref, logits_ref, anom_ref, ztopo_ref, zsem2_ref):
    d = degp_ref[...]
    deg = d[0, :, :1] + d[1, :, :1] + 1.0
    dinv = lax.rsqrt(deg)
    a = acc_ref[...]
    pre = a[0] + a[1]
    h = jnp.maximum(pre * dinv + bg_ref[...], 0.0)
    zt = jnp.dot(h, wpt_ref[...], preferred_element_type=jnp.float32) + bpt_ref[...]
    ztopo_ref[...] = zt
    logits_ref[...] = (
        jnp.dot(zt, wcls_ref[...], preferred_element_type=jnp.float32)
        + bcls_ref[...]
    )
    zs = zsem_ref[...]
    zsem2_ref[...] = zs
    diff = zt - zs
    anom_ref[...] = jnp.sqrt(jnp.sum(diff * diff, axis=1, keepdims=True))


def kernel(x, edge_index, W_gcn, b_gcn, W_pt, b_pt, W_ps, b_ps, W_cls, b_cls):
    n, in_dim = x.shape
    hid = W_gcn.shape[1]
    al = W_pt.shape[1]
    ncls = W_cls.shape[1]
    e = edge_index.shape[1]

    chunks = -(-e // (NW * CHUNK))
    e_pad = NW * chunks * CHUNK
    n_pad = n + NS

    pad = e_pad - e
    src_p = jnp.concatenate(
        [edge_index[0], jnp.zeros((pad,), jnp.int32)]).reshape(NW, chunks, CHUNK)
    dst_p = jnp.concatenate(
        [edge_index[1], jnp.full((pad,), n, jnp.int32)]).reshape(NW, chunks, CHUNK)

    zeros_deg = jnp.zeros((n_pad // NS, DEG_W), jnp.float32)
    ones_blk = jnp.ones((CHUNK, DEG_W), jnp.float32)
    zeros_acc = jnp.zeros((n_pad // NS, hid), jnp.float32)

    deg_fn = pl.kernel(
        functools.partial(_deg_body, n=n, n_pad=n_pad, chunks=chunks),
        out_type=jax.ShapeDtypeStruct((NC, n_pad, DEG_W), jnp.float32),
        mesh=_sc_mesh(),
        scratch_types=[
            pltpu.VMEM((chunks, CHUNK), jnp.int32),
            pltpu.VMEM((CHUNK, DEG_W), jnp.float32),
            *[pltpu.SemaphoreType.DMA for _ in range(PIPE_G)],
            pltpu.VMEM_SHARED((n_pad, DEG_W), jnp.float32),
        ],
        compiler_params=pltpu.CompilerParams(use_tc_tiling_on_sc=False),
    )
    deg_p = deg_fn(dst_p, zeros_deg, ones_blk)

    blk = 2000
    grid = (n // blk,)
    y, z_sem = pl.pallas_call(
        _tc_a_body,
        grid=grid,
        in_specs=[
            pl.BlockSpec((blk, in_dim), lambda i: (i, 0)),
            pl.BlockSpec((NC, blk, DEG_W), lambda i: (0, i, 0)),
            pl.BlockSpec((in_dim, hid), lambda i: (0, 0)),
            pl.BlockSpec((in_dim, al), lambda i: (0, 0)),
            pl.BlockSpec((1, al), lambda i: (0, 0)),
        ],
        out_specs=[
            pl.BlockSpec((blk, hid), lambda i: (i, 0)),
            pl.BlockSpec((blk, al), lambda i: (i, 0)),
        ],
        out_shape=[
            jax.ShapeDtypeStruct((n, hid), jnp.float32),
            jax.ShapeDtypeStruct((n, al), jnp.float32),
        ],
    )(x, deg_p, W_gcn, W_ps, b_ps.reshape(1, al))

    acc_fn = pl.kernel(
        functools.partial(_edge_body, n=n, n_pad=n_pad, chunks=chunks, hid=hid),
        out_type=jax.ShapeDtypeStruct((NC, n_pad, hid), jnp.float32),
        mesh=_sc_mesh(),
        scratch_types=[
            pltpu.VMEM((chunks, CHUNK), jnp.int32),
            pltpu.VMEM((chunks, CHUNK), jnp.int32),
            *[pltpu.VMEM((CHUNK, hid), jnp.float32) for _ in range(PIPE_G)],
            *[pltpu.SemaphoreType.DMA for _ in range(PIPE_G)],
            pltpu.VMEM_SHARED((n_pad, hid), jnp.float32),
            pltpu.VMEM_SHARED((n_pad, hid), jnp.float32),
        ],
        compiler_params=pltpu.CompilerParams(use_tc_tiling_on_sc=False),
    )
    acc = acc_fn(src_p, dst_p, y, zeros_acc)

    logits, anom, z_topo, z_sem_out = pl.pallas_call(
        _tc_e_body,
        grid=grid,
        in_specs=[
            pl.BlockSpec((NC, blk, hid), lambda i: (0, i, 0)),
            pl.BlockSpec((NC, blk, DEG_W), lambda i: (0, i, 0)),
            pl.BlockSpec((blk, al), lambda i: (i, 0)),
            pl.BlockSpec((1, hid), lambda i: (0, 0)),
            pl.BlockSpec((hid, al), lambda i: (0, 0)),
            pl.BlockSpec((1, al), lambda i: (0, 0)),
            pl.BlockSpec((al, ncls), lambda i: (0, 0)),
            pl.BlockSpec((1, ncls), lambda i: (0, 0)),
        ],
        out_specs=[
            pl.BlockSpec((blk, ncls), lambda i: (i, 0)),
            pl.BlockSpec((blk, 1), lambda i: (i, 0)),
            pl.BlockSpec((blk, al), lambda i: (i, 0)),
            pl.BlockSpec((blk, al), lambda i: (i, 0)),
        ],
        out_shape=[
            jax.ShapeDtypeStruct((n, ncls), jnp.float32),
            jax.ShapeDtypeStruct((n, 1), jnp.float32),
            jax.ShapeDtypeStruct((n, al), jnp.float32),
            jax.ShapeDtypeStruct((n, al), jnp.float32),
        ],
    )(acc, deg_p, z_sem, b_gcn.reshape(1, hid), W_pt, b_pt.reshape(1, al),
      W_cls, b_cls.reshape(1, ncls))

    return (logits, anom.reshape(n), z_topo, z_sem_out)


# speedup vs baseline: 1.0727x; 1.0727x over previous
import functools

import jax
import jax.numpy as jnp
from jax import lax
from jax.experimental import pallas as pl
from jax.experimental.pallas import tpu as pltpu
from jax.experimental.pallas import tpu_sc as plsc

NC = 2
NS = 16
NW = NC * NS
CHUNK = 80
DEG_W = 16
PIPE_G = 6


def _sc_mesh():
    return plsc.VectorSubcoreMesh(
        core_axis_name="c", subcore_axis_name="s", num_cores=NC, num_subcores=NS
    )


def _deg_body(dst_hbm, zeros_hbm, ones_hbm, deg_out, idx_v, ones_v, *scratch,
              n, n_pad, chunks):
    sems = scratch[:PIPE_G]
    deg_sh = scratch[PIPE_G]
    c = lax.axis_index("c")
    s = lax.axis_index("s")
    w = s * NC + c
    rpt = n_pad // NS
    pltpu.sync_copy(zeros_hbm, deg_sh.at[pl.ds(s * rpt, rpt)])
    pltpu.sync_copy(dst_hbm.at[w], idx_v)
    pltpu.sync_copy(ones_hbm, ones_v)
    plsc.subcore_barrier()

    ng = chunks // PIPE_G
    rem = chunks - ng * PIPE_G

    def group(base, count):
        descs = []
        for i in range(count):
            descs.append(
                pltpu.async_copy(ones_v, deg_sh.at[idx_v.at[base + i]],
                                 sems[i], add=True)
            )
        for i in range(count):
            descs[i].wait()

    def body(g, carry):
        group(g * PIPE_G, PIPE_G)
        return carry

    lax.fori_loop(0, ng, body, 0)
    if rem:
        group(ng * PIPE_G, rem)
    plsc.subcore_barrier()
    pltpu.sync_copy(deg_sh.at[pl.ds(s * rpt, rpt)],
                    deg_out.at[c, pl.ds(s * rpt, rpt)])


def _edge_body(src_hbm, dst_hbm, y_hbm, zeros_hbm, acc_out,
               src_v, dst_v, *scratch, n, n_pad, chunks, hid):
    rows = scratch[:PIPE_G]
    sems = scratch[PIPE_G:2 * PIPE_G]
    y_sh, acc_sh = scratch[2 * PIPE_G:]
    c = lax.axis_index("c")
    s = lax.axis_index("s")
    w = s * NC + c
    rpt = n_pad // NS
    spt = n // NS
    pltpu.sync_copy(y_hbm.at[pl.ds(s * spt, spt)], y_sh.at[pl.ds(s * spt, spt)])

    @pl.when(c == 0)
    def _():
        pltpu.sync_copy(y_hbm.at[pl.ds(s * spt, spt)],
                        acc_sh.at[pl.ds(s * spt, spt)])

    @pl.when(c != 0)
    def _():
        pltpu.sync_copy(zeros_hbm, acc_sh.at[pl.ds(s * rpt, rpt)])

    pltpu.sync_copy(src_hbm.at[w], src_v)
    pltpu.sync_copy(dst_hbm.at[w], dst_v)
    plsc.subcore_barrier()

    ng = chunks // PIPE_G
    rem = chunks - ng * PIPE_G

    def group(base, count):
        gd = []
        for i in range(count):
            gd.append(
                pltpu.async_copy(y_sh.at[src_v.at[base + i]], rows[i], sems[i])
            )
        for i in range(count):
            gd[i].wait()
            pltpu.sync_copy(rows[i], acc_sh.at[dst_v.at[base + i]], add=True)

    def body(g, carry):
        group(g * PIPE_G, PIPE_G)
        return carry

    lax.fori_loop(0, ng, body, 0)
    if rem:
        group(ng * PIPE_G, rem)
    plsc.subcore_barrier()
    pltpu.sync_copy(acc_sh.at[pl.ds(s * rpt, rpt)],
                    acc_out.at[c, pl.ds(s * rpt, rpt)])


def _tc_x_body(x_ref, wg_ref, wps_ref, bps_ref, xw_ref, zsem_ref):
    xw_ref[...] = jnp.dot(x_ref[...], wg_ref[...],
                          preferred_element_type=jnp.float32)
    zsem_ref[...] = (
        jnp.dot(x_ref[...], wps_ref[...], preferred_element_type=jnp.float32)
        + bps_ref[...]
    )


def _tc_y_body(xw_ref, degp_ref, y_ref):
    d = degp_ref[...]
    deg = d[0, :, :1] + d[1, :, :1] + 1.0
    dinv = lax.rsqrt(deg)
    y_ref[...] = xw_ref[...] * dinv


def _tc_e_body(acc_ref, degp_ref, zsem_ref, bg_ref, wpt_ref, bpt_ref,
               wcls_ref, bcls_ref, logits_ref, anom_ref, ztopo_ref, zsem2_ref):
    d = degp_ref[...]
    deg = d[0, :, :1] + d[1, :, :1] + 1.0
    dinv = lax.rsqrt(deg)
    a = acc_ref[...]
    pre = a[0] + a[1]
    h = jnp.maximum(pre * dinv + bg_ref[...], 0.0)
    zt = jnp.dot(h, wpt_ref[...], preferred_element_type=jnp.float32) + bpt_ref[...]
    ztopo_ref[...] = zt
    logits_ref[...] = (
        jnp.dot(zt, wcls_ref[...], preferred_element_type=jnp.float32)
        + bcls_ref[...]
    )
    zs = zsem_ref[...]
    zsem2_ref[...] = zs
    diff = zt - zs
    anom_ref[...] = jnp.sqrt(jnp.sum(diff * diff, axis=1, keepdims=True))


def kernel(x, edge_index, W_gcn, b_gcn, W_pt, b_pt, W_ps, b_ps, W_cls, b_cls):
    n, in_dim = x.shape
    hid = W_gcn.shape[1]
    al = W_pt.shape[1]
    ncls = W_cls.shape[1]
    e = edge_index.shape[1]

    chunks = e // (NW * CHUNK)
    n_pad = n

    src_p = edge_index[0].reshape(NW, chunks, CHUNK)
    dst_p = edge_index[1].reshape(NW, chunks, CHUNK)

    zeros_deg = jnp.zeros((n_pad // NS, DEG_W), jnp.float32)
    ones_blk = jnp.ones((CHUNK, DEG_W), jnp.float32)
    zeros_acc = jnp.zeros((n_pad // NS, hid), jnp.float32)

    deg_fn = pl.kernel(
        functools.partial(_deg_body, n=n, n_pad=n_pad, chunks=chunks),
        out_type=jax.ShapeDtypeStruct((NC, n_pad, DEG_W), jnp.float32),
        mesh=_sc_mesh(),
        scratch_types=[
            pltpu.VMEM((chunks, CHUNK), jnp.int32),
            pltpu.VMEM((CHUNK, DEG_W), jnp.float32),
            *[pltpu.SemaphoreType.DMA for _ in range(PIPE_G)],
            pltpu.VMEM_SHARED((n_pad, DEG_W), jnp.float32),
        ],
        compiler_params=pltpu.CompilerParams(use_tc_tiling_on_sc=False),
    )
    deg_p = deg_fn(dst_p, zeros_deg, ones_blk)

    blk = 2000
    grid = (n // blk,)
    xw, z_sem = pl.pallas_call(
        _tc_x_body,
        grid=grid,
        in_specs=[
            pl.BlockSpec((blk, in_dim), lambda i: (i, 0)),
            pl.BlockSpec((in_dim, hid), lambda i: (0, 0)),
            pl.BlockSpec((in_dim, al), lambda i: (0, 0)),
            pl.BlockSpec((1, al), lambda i: (0, 0)),
        ],
        out_specs=[
            pl.BlockSpec((blk, hid), lambda i: (i, 0)),
            pl.BlockSpec((blk, al), lambda i: (i, 0)),
        ],
        out_shape=[
            jax.ShapeDtypeStruct((n, hid), jnp.float32),
            jax.ShapeDtypeStruct((n, al), jnp.float32),
        ],
    )(x, W_gcn, W_ps, b_ps.reshape(1, al))

    y = pl.pallas_call(
        _tc_y_body,
        grid=grid,
        in_specs=[
            pl.BlockSpec((blk, hid), lambda i: (i, 0)),
            pl.BlockSpec((NC, blk, DEG_W), lambda i: (0, i, 0)),
        ],
        out_specs=pl.BlockSpec((blk, hid), lambda i: (i, 0)),
        out_shape=jax.ShapeDtypeStruct((n, hid), jnp.float32),
    )(xw, deg_p)

    acc_fn = pl.kernel(
        functools.partial(_edge_body, n=n, n_pad=n_pad, chunks=chunks, hid=hid),
        out_type=jax.ShapeDtypeStruct((NC, n_pad, hid), jnp.float32),
        mesh=_sc_mesh(),
        scratch_types=[
            pltpu.VMEM((chunks, CHUNK), jnp.int32),
            pltpu.VMEM((chunks, CHUNK), jnp.int32),
            *[pltpu.VMEM((CHUNK, hid), jnp.float32) for _ in range(PIPE_G)],
            *[pltpu.SemaphoreType.DMA for _ in range(PIPE_G)],
            pltpu.VMEM_SHARED((n_pad, hid), jnp.float32),
            pltpu.VMEM_SHARED((n_pad, hid), jnp.float32),
        ],
        compiler_params=pltpu.CompilerParams(use_tc_tiling_on_sc=False),
    )
    acc = acc_fn(src_p, dst_p, y, zeros_acc)

    logits, anom, z_topo, z_sem_out = pl.pallas_call(
        _tc_e_body,
        grid=grid,
        in_specs=[
            pl.BlockSpec((NC, blk, hid), lambda i: (0, i, 0)),
            pl.BlockSpec((NC, blk, DEG_W), lambda i: (0, i, 0)),
            pl.BlockSpec((blk, al), lambda i: (i, 0)),
            pl.BlockSpec((1, hid), lambda i: (0, 0)),
            pl.BlockSpec((hid, al), lambda i: (0, 0)),
            pl.BlockSpec((1, al), lambda i: (0, 0)),
            pl.BlockSpec((al, ncls), lambda i: (0, 0)),
            pl.BlockSpec((1, ncls), lambda i: (0, 0)),
        ],
        out_specs=[
            pl.BlockSpec((blk, ncls), lambda i: (i, 0)),
            pl.BlockSpec((blk, 1), lambda i: (i, 0)),
            pl.BlockSpec((blk, al), lambda i: (i, 0)),
            pl.BlockSpec((blk, al), lambda i: (i, 0)),
        ],
        out_shape=[
            jax.ShapeDtypeStruct((n, ncls), jnp.float32),
            jax.ShapeDtypeStruct((n, 1), jnp.float32),
            jax.ShapeDtypeStruct((n, al), jnp.float32),
            jax.ShapeDtypeStruct((n, al), jnp.float32),
        ],
    )(acc, deg_p, z_sem, b_gcn.reshape(1, hid), W_pt, b_pt.reshape(1, al),
      W_cls, b_cls.reshape(1, ncls))

    return (logits, anom.reshape(n), z_topo, z_sem_out)


# speedup vs baseline: 1.1393x; 1.0621x over previous
import functools

import jax
import jax.numpy as jnp
from jax import lax
from jax.experimental import pallas as pl
from jax.experimental.pallas import tpu as pltpu
from jax.experimental.pallas import tpu_sc as plsc

NC = 2
NS = 16
NW = NC * NS
CHUNK = 80
DEG_W = 16
PIPE_G = 6


def _sc_mesh():
    return plsc.VectorSubcoreMesh(
        core_axis_name="c", subcore_axis_name="s", num_cores=NC, num_subcores=NS
    )


def _deg_body(edge_hbm, zeros_hbm, ones_hbm, deg_out, idx_v, ones_v, *scratch,
              n, n_pad, chunks):
    sems = scratch[:PIPE_G]
    deg_sh = scratch[PIPE_G]
    c = lax.axis_index("c")
    s = lax.axis_index("s")
    w = s * NC + c
    rpt = n_pad // NS
    tpe = chunks * CHUNK
    pltpu.sync_copy(zeros_hbm, deg_sh.at[pl.ds(s * rpt, rpt)])
    pltpu.sync_copy(edge_hbm.at[1, pl.ds(w * tpe, tpe)], idx_v)
    pltpu.sync_copy(ones_hbm, ones_v)
    plsc.subcore_barrier()

    ng = chunks // PIPE_G
    rem = chunks - ng * PIPE_G

    def group(base, count):
        descs = []
        for i in range(count):
            descs.append(
                pltpu.async_copy(
                    ones_v,
                    deg_sh.at[idx_v.at[pl.ds((base + i) * CHUNK, CHUNK)]],
                    sems[i], add=True)
            )
        for i in range(count):
            descs[i].wait()

    def body(g, carry):
        group(g * PIPE_G, PIPE_G)
        return carry

    lax.fori_loop(0, ng, body, 0)
    if rem:
        group(ng * PIPE_G, rem)
    plsc.subcore_barrier()
    pltpu.sync_copy(deg_sh.at[pl.ds(s * rpt, rpt)],
                    deg_out.at[c, pl.ds(s * rpt, rpt)])


def _edge_body(edge_hbm, y_hbm, zeros_hbm, acc_out,
               src_v, dst_v, *scratch, n, n_pad, chunks, hid):
    rows = scratch[:PIPE_G]
    sems = scratch[PIPE_G:2 * PIPE_G]
    y_sh, acc_sh = scratch[2 * PIPE_G:]
    c = lax.axis_index("c")
    s = lax.axis_index("s")
    w = s * NC + c
    rpt = n_pad // NS
    spt = n // NS
    pltpu.sync_copy(y_hbm.at[pl.ds(s * spt, spt)], y_sh.at[pl.ds(s * spt, spt)])

    @pl.when(c == 0)
    def _():
        pltpu.sync_copy(y_hbm.at[pl.ds(s * spt, spt)],
                        acc_sh.at[pl.ds(s * spt, spt)])

    @pl.when(c != 0)
    def _():
        pltpu.sync_copy(zeros_hbm, acc_sh.at[pl.ds(s * rpt, rpt)])

    tpe = chunks * CHUNK
    pltpu.sync_copy(edge_hbm.at[0, pl.ds(w * tpe, tpe)], src_v)
    pltpu.sync_copy(edge_hbm.at[1, pl.ds(w * tpe, tpe)], dst_v)
    plsc.subcore_barrier()

    ng = chunks // PIPE_G
    rem = chunks - ng * PIPE_G

    def group(base, count):
        gd = []
        for i in range(count):
            gd.append(
                pltpu.async_copy(
                    y_sh.at[src_v.at[pl.ds((base + i) * CHUNK, CHUNK)]],
                    rows[i], sems[i])
            )
        for i in range(count):
            gd[i].wait()
            pltpu.sync_copy(
                rows[i],
                acc_sh.at[dst_v.at[pl.ds((base + i) * CHUNK, CHUNK)]],
                add=True)

    def body(g, carry):
        group(g * PIPE_G, PIPE_G)
        return carry

    lax.fori_loop(0, ng, body, 0)
    if rem:
        group(ng * PIPE_G, rem)
    plsc.subcore_barrier()
    pltpu.sync_copy(acc_sh.at[pl.ds(s * rpt, rpt)],
                    acc_out.at[c, pl.ds(s * rpt, rpt)])


def _tc_x_body(x_ref, wg_ref, wps_ref, bps_ref, xw_ref, zsem_ref):
    xw_ref[...] = jnp.dot(x_ref[...], wg_ref[...],
                          preferred_element_type=jnp.float32)
    zsem_ref[...] = (
        jnp.dot(x_ref[...], wps_ref[...], preferred_element_type=jnp.float32)
        + bps_ref[...]
    )


def _tc_y_body(xw_ref, degp_ref, y_ref):
    d = degp_ref[...]
    deg = d[0, :, :1] + d[1, :, :1] + 1.0
    dinv = lax.rsqrt(deg)
    y_ref[...] = xw_ref[...] * dinv


def _tc_e_body(acc_ref, degp_ref, zsem_ref, bg_ref, wpt_ref, bpt_ref,
               wcls_ref, bcls_ref, logits_ref, anom_ref, ztopo_ref, zsem2_ref):
    d = degp_ref[...]
    deg = d[0, :, :1] + d[1, :, :1] + 1.0
    dinv = lax.rsqrt(deg)
    a = acc_ref[...]
    pre = a[0] + a[1]
    h = jnp.maximum(pre * dinv + bg_ref[...], 0.0)
    zt = jnp.dot(h, wpt_ref[...], preferred_element_type=jnp.float32) + bpt_ref[...]
    ztopo_ref[...] = zt
    logits_ref[...] = (
        jnp.dot(zt, wcls_ref[...], preferred_element_type=jnp.float32)
        + bcls_ref[...]
    )
    zs = zsem_ref[...]
    zsem2_ref[...] = zs
    diff = zt - zs
    anom_ref[...] = jnp.sqrt(jnp.sum(diff * diff, axis=1, keepdims=True))


def kernel(x, edge_index, W_gcn, b_gcn, W_pt, b_pt, W_ps, b_ps, W_cls, b_cls):
    n, in_dim = x.shape
    hid = W_gcn.shape[1]
    al = W_pt.shape[1]
    ncls = W_cls.shape[1]
    e = edge_index.shape[1]

    chunks = e // (NW * CHUNK)
    n_pad = n
    tpe = chunks * CHUNK

    zeros_deg = jnp.zeros((n_pad // NS, DEG_W), jnp.float32)
    ones_blk = jnp.ones((CHUNK, DEG_W), jnp.float32)
    zeros_acc = jnp.zeros((n_pad // NS, hid), jnp.float32)

    deg_fn = pl.kernel(
        functools.partial(_deg_body, n=n, n_pad=n_pad, chunks=chunks),
        out_type=jax.ShapeDtypeStruct((NC, n_pad, DEG_W), jnp.float32),
        mesh=_sc_mesh(),
        scratch_types=[
            pltpu.VMEM((tpe,), jnp.int32),
            pltpu.VMEM((CHUNK, DEG_W), jnp.float32),
            *[pltpu.SemaphoreType.DMA for _ in range(PIPE_G)],
            pltpu.VMEM_SHARED((n_pad, DEG_W), jnp.float32),
        ],
        compiler_params=pltpu.CompilerParams(use_tc_tiling_on_sc=False),
    )
    deg_p = deg_fn(edge_index, zeros_deg, ones_blk)

    blk = 2000
    grid = (n // blk,)
    xw, z_sem = pl.pallas_call(
        _tc_x_body,
        grid=grid,
        in_specs=[
            pl.BlockSpec((blk, in_dim), lambda i: (i, 0)),
            pl.BlockSpec((in_dim, hid), lambda i: (0, 0)),
            pl.BlockSpec((in_dim, al), lambda i: (0, 0)),
            pl.BlockSpec((1, al), lambda i: (0, 0)),
        ],
        out_specs=[
            pl.BlockSpec((blk, hid), lambda i: (i, 0)),
            pl.BlockSpec((blk, al), lambda i: (i, 0)),
        ],
        out_shape=[
            jax.ShapeDtypeStruct((n, hid), jnp.float32),
            jax.ShapeDtypeStruct((n, al), jnp.float32),
        ],
    )(x, W_gcn, W_ps, b_ps.reshape(1, al))

    y = pl.pallas_call(
        _tc_y_body,
        grid=grid,
        in_specs=[
            pl.BlockSpec((blk, hid), lambda i: (i, 0)),
            pl.BlockSpec((NC, blk, DEG_W), lambda i: (0, i, 0)),
        ],
        out_specs=pl.BlockSpec((blk, hid), lambda i: (i, 0)),
        out_shape=jax.ShapeDtypeStruct((n, hid), jnp.float32),
    )(xw, deg_p)

    acc_fn = pl.kernel(
        functools.partial(_edge_body, n=n, n_pad=n_pad, chunks=chunks, hid=hid),
        out_type=jax.ShapeDtypeStruct((NC, n_pad, hid), jnp.float32),
        mesh=_sc_mesh(),
        scratch_types=[
            pltpu.VMEM((tpe,), jnp.int32),
            pltpu.VMEM((tpe,), jnp.int32),
            *[pltpu.VMEM((CHUNK, hid), jnp.float32) for _ in range(PIPE_G)],
            *[pltpu.SemaphoreType.DMA for _ in range(PIPE_G)],
            pltpu.VMEM_SHARED((n_pad, hid), jnp.float32),
            pltpu.VMEM_SHARED((n_pad, hid), jnp.float32),
        ],
        compiler_params=pltpu.CompilerParams(use_tc_tiling_on_sc=False),
    )
    acc = acc_fn(edge_index, y, zeros_acc)

    logits, anom, z_topo, z_sem_out = pl.pallas_call(
        _tc_e_body,
        grid=grid,
        in_specs=[
            pl.BlockSpec((NC, blk, hid), lambda i: (0, i, 0)),
            pl.BlockSpec((NC, blk, DEG_W), lambda i: (0, i, 0)),
            pl.BlockSpec((blk, al), lambda i: (i, 0)),
            pl.BlockSpec((1, hid), lambda i: (0, 0)),
            pl.BlockSpec((hid, al), lambda i: (0, 0)),
            pl.BlockSpec((1, al), lambda i: (0, 0)),
            pl.BlockSpec((al, ncls), lambda i: (0, 0)),
            pl.BlockSpec((1, ncls), lambda i: (0, 0)),
        ],
        out_specs=[
            pl.BlockSpec((blk, ncls), lambda i: (i, 0)),
            pl.BlockSpec((blk, 1), lambda i: (i, 0)),
            pl.BlockSpec((blk, al), lambda i: (i, 0)),
            pl.BlockSpec((blk, al), lambda i: (i, 0)),
        ],
        out_shape=[
            jax.ShapeDtypeStruct((n, ncls), jnp.float32),
            jax.ShapeDtypeStruct((n, 1), jnp.float32),
            jax.ShapeDtypeStruct((n, al), jnp.float32),
            jax.ShapeDtypeStruct((n, al), jnp.float32),
        ],
    )(acc, deg_p, z_sem, b_gcn.reshape(1, hid), W_pt, b_pt.reshape(1, al),
      W_cls, b_cls.reshape(1, ncls))

    return (logits, anom.reshape(n), z_topo, z_sem_out)


# speedup vs baseline: 1.1795x; 1.0352x over previous
import functools

import jax
import jax.numpy as jnp
from jax import lax
from jax.experimental import pallas as pl
from jax.experimental.pallas import tpu as pltpu
from jax.experimental.pallas import tpu_sc as plsc

NC = 2
NS = 16
NW = NC * NS
CHUNK = 80
DEG_W = 16
PIPE_G = 6


def _sc_mesh():
    return plsc.VectorSubcoreMesh(
        core_axis_name="c", subcore_axis_name="s", num_cores=NC, num_subcores=NS
    )


def _deg_body(edge_hbm, zeros_hbm, ones_hbm, deg_out, idx_v, ones_v, *scratch,
              n, n_pad, chunks):
    sems = scratch[:PIPE_G]
    deg_sh = scratch[PIPE_G]
    c = lax.axis_index("c")
    s = lax.axis_index("s")
    w = s * NC + c
    rpt = n_pad // NS
    tpe = chunks * CHUNK
    pltpu.sync_copy(zeros_hbm, deg_sh.at[pl.ds(s * rpt, rpt)])
    pltpu.sync_copy(edge_hbm.at[1, pl.ds(w * tpe, tpe)], idx_v)
    pltpu.sync_copy(ones_hbm, ones_v)
    plsc.subcore_barrier()

    ng = chunks // PIPE_G
    rem = chunks - ng * PIPE_G

    def group(base, count):
        descs = []
        for i in range(count):
            descs.append(
                pltpu.async_copy(
                    ones_v,
                    deg_sh.at[idx_v.at[pl.ds((base + i) * CHUNK, CHUNK)]],
                    sems[i], add=True)
            )
        for i in range(count):
            descs[i].wait()

    def body(g, carry):
        group(g * PIPE_G, PIPE_G)
        return carry

    lax.fori_loop(0, ng, body, 0)
    if rem:
        group(ng * PIPE_G, rem)
    plsc.subcore_barrier()
    pltpu.sync_copy(deg_sh.at[pl.ds(s * rpt, rpt)],
                    deg_out.at[c, pl.ds(s * rpt, rpt)])


def _edge_body(edge_hbm, y_hbm, zeros_hbm, acc_out,
               src_v, dst_v, *scratch, n, n_pad, chunks, hid):
    rows = scratch[:PIPE_G]
    sems = scratch[PIPE_G:2 * PIPE_G]
    y_sh, acc_sh = scratch[2 * PIPE_G:]
    c = lax.axis_index("c")
    s = lax.axis_index("s")
    w = s * NC + c
    rpt = n_pad // NS
    spt = n // NS
    pltpu.sync_copy(y_hbm.at[pl.ds(s * spt, spt)], y_sh.at[pl.ds(s * spt, spt)])

    @pl.when(c == 0)
    def _():
        pltpu.sync_copy(y_hbm.at[pl.ds(s * spt, spt)],
                        acc_sh.at[pl.ds(s * spt, spt)])

    @pl.when(c != 0)
    def _():
        pltpu.sync_copy(zeros_hbm, acc_sh.at[pl.ds(s * rpt, rpt)])

    tpe = chunks * CHUNK
    pltpu.sync_copy(edge_hbm.at[0, pl.ds(w * tpe, tpe)], src_v)
    pltpu.sync_copy(edge_hbm.at[1, pl.ds(w * tpe, tpe)], dst_v)
    plsc.subcore_barrier()

    ng = chunks // PIPE_G
    rem = chunks - ng * PIPE_G

    def group(base, count):
        gd = []
        for i in range(count):
            gd.append(
                pltpu.async_copy(
                    y_sh.at[src_v.at[pl.ds((base + i) * CHUNK, CHUNK)]],
                    rows[i], sems[i])
            )
        for i in range(count):
            gd[i].wait()
            pltpu.sync_copy(
                rows[i],
                acc_sh.at[dst_v.at[pl.ds((base + i) * CHUNK, CHUNK)]],
                add=True)

    def body(g, carry):
        group(g * PIPE_G, PIPE_G)
        return carry

    lax.fori_loop(0, ng, body, 0)
    if rem:
        group(ng * PIPE_G, rem)
    plsc.subcore_barrier()
    pltpu.sync_copy(acc_sh.at[pl.ds(s * rpt, rpt)],
                    acc_out.at[c, pl.ds(s * rpt, rpt)])


def _tc_x_body(x_ref, wg_ref, wps_ref, bps_ref, xw_ref, zsem_ref):
    xw_ref[...] = jnp.dot(x_ref[...], wg_ref[...],
                          preferred_element_type=jnp.float32)
    zsem_ref[...] = (
        jnp.dot(x_ref[...], wps_ref[...], preferred_element_type=jnp.float32)
        + bps_ref[...]
    )


def _tc_y_body(xw_ref, degp_ref, y_ref):
    d = degp_ref[...]
    deg = d[0, :, :1] + d[1, :, :1] + 1.0
    dinv = lax.rsqrt(deg)
    y_ref[...] = xw_ref[...] * dinv


def _tc_e_body(acc_ref, degp_ref, zsem_ref, bg_ref, wpt_ref, bpt_ref,
               wcls_ref, bcls_ref, logits_ref, anom_ref, ztopo_ref):
    d = degp_ref[...]
    deg = d[0, :, :1] + d[1, :, :1] + 1.0
    dinv = lax.rsqrt(deg)
    a = acc_ref[...]
    pre = a[0] + a[1]
    h = jnp.maximum(pre * dinv + bg_ref[...], 0.0)
    zt = jnp.dot(h, wpt_ref[...], preferred_element_type=jnp.float32) + bpt_ref[...]
    ztopo_ref[...] = zt
    logits_ref[...] = (
        jnp.dot(zt, wcls_ref[...], preferred_element_type=jnp.float32)
        + bcls_ref[...]
    )
    diff = zt - zsem_ref[...]
    anom_ref[...] = jnp.sqrt(jnp.sum(diff * diff, axis=1, keepdims=True))


def kernel(x, edge_index, W_gcn, b_gcn, W_pt, b_pt, W_ps, b_ps, W_cls, b_cls):
    n, in_dim = x.shape
    hid = W_gcn.shape[1]
    al = W_pt.shape[1]
    ncls = W_cls.shape[1]
    e = edge_index.shape[1]

    chunks = e // (NW * CHUNK)
    n_pad = n
    tpe = chunks * CHUNK

    zeros_deg = jnp.zeros((n_pad // NS, DEG_W), jnp.float32)
    ones_blk = jnp.ones((CHUNK, DEG_W), jnp.float32)
    zeros_acc = jnp.zeros((n_pad // NS, hid), jnp.float32)

    deg_fn = pl.kernel(
        functools.partial(_deg_body, n=n, n_pad=n_pad, chunks=chunks),
        out_type=jax.ShapeDtypeStruct((NC, n_pad, DEG_W), jnp.float32),
        mesh=_sc_mesh(),
        scratch_types=[
            pltpu.VMEM((tpe,), jnp.int32),
            pltpu.VMEM((CHUNK, DEG_W), jnp.float32),
            *[pltpu.SemaphoreType.DMA for _ in range(PIPE_G)],
            pltpu.VMEM_SHARED((n_pad, DEG_W), jnp.float32),
        ],
        compiler_params=pltpu.CompilerParams(use_tc_tiling_on_sc=False),
    )
    deg_p = deg_fn(edge_index, zeros_deg, ones_blk)

    blk = 2000
    grid = (n // blk,)
    xw, z_sem = pl.pallas_call(
        _tc_x_body,
        grid=grid,
        in_specs=[
            pl.BlockSpec((blk, in_dim), lambda i: (i, 0)),
            pl.BlockSpec((in_dim, hid), lambda i: (0, 0)),
            pl.BlockSpec((in_dim, al), lambda i: (0, 0)),
            pl.BlockSpec((1, al), lambda i: (0, 0)),
        ],
        out_specs=[
            pl.BlockSpec((blk, hid), lambda i: (i, 0)),
            pl.BlockSpec((blk, al), lambda i: (i, 0)),
        ],
        out_shape=[
            jax.ShapeDtypeStruct((n, hid), jnp.float32),
            jax.ShapeDtypeStruct((n, al), jnp.float32),
        ],
    )(x, W_gcn, W_ps, b_ps.reshape(1, al))

    y = pl.pallas_call(
        _tc_y_body,
        grid=grid,
        in_specs=[
            pl.BlockSpec((blk, hid), lambda i: (i, 0)),
            pl.BlockSpec((NC, blk, DEG_W), lambda i: (0, i, 0)),
        ],
        out_specs=pl.BlockSpec((blk, hid), lambda i: (i, 0)),
        out_shape=jax.ShapeDtypeStruct((n, hid), jnp.float32),
    )(xw, deg_p)

    acc_fn = pl.kernel(
        functools.partial(_edge_body, n=n, n_pad=n_pad, chunks=chunks, hid=hid),
        out_type=jax.ShapeDtypeStruct((NC, n_pad, hid), jnp.float32),
        mesh=_sc_mesh(),
        scratch_types=[
            pltpu.VMEM((tpe,), jnp.int32),
            pltpu.VMEM((tpe,), jnp.int32),
            *[pltpu.VMEM((CHUNK, hid), jnp.float32) for _ in range(PIPE_G)],
            *[pltpu.SemaphoreType.DMA for _ in range(PIPE_G)],
            pltpu.VMEM_SHARED((n_pad, hid), jnp.float32),
            pltpu.VMEM_SHARED((n_pad, hid), jnp.float32),
        ],
        compiler_params=pltpu.CompilerParams(use_tc_tiling_on_sc=False),
    )
    acc = acc_fn(edge_index, y, zeros_acc)

    logits, anom, z_topo = pl.pallas_call(
        _tc_e_body,
        grid=grid,
        in_specs=[
            pl.BlockSpec((NC, blk, hid), lambda i: (0, i, 0)),
            pl.BlockSpec((NC, blk, DEG_W), lambda i: (0, i, 0)),
            pl.BlockSpec((blk, al), lambda i: (i, 0)),
            pl.BlockSpec((1, hid), lambda i: (0, 0)),
            pl.BlockSpec((hid, al), lambda i: (0, 0)),
            pl.BlockSpec((1, al), lambda i: (0, 0)),
            pl.BlockSpec((al, ncls), lambda i: (0, 0)),
            pl.BlockSpec((1, ncls), lambda i: (0, 0)),
        ],
        out_specs=[
            pl.BlockSpec((blk, ncls), lambda i: (i, 0)),
            pl.BlockSpec((blk, 1), lambda i: (i, 0)),
            pl.BlockSpec((blk, al), lambda i: (i, 0)),
        ],
        out_shape=[
            jax.ShapeDtypeStruct((n, ncls), jnp.float32),
            jax.ShapeDtypeStruct((n, 1), jnp.float32),
            jax.ShapeDtypeStruct((n, al), jnp.float32),
        ],
    )(acc, deg_p, z_sem, b_gcn.reshape(1, hid), W_pt, b_pt.reshape(1, al),
      W_cls, b_cls.reshape(1, ncls))

    return (logits, anom.reshape(n), z_topo, z_sem)
